# baseline (speedup 1.0000x reference)
import functools
import jax
import jax.numpy as jnp
from jax import lax
from jax.experimental import pallas as pl
from jax.experimental.pallas import tpu as pltpu
from jax.experimental.pallas import tpu_sc as plsc

NCORE = 2
NSUB = 16
NW = NCORE * NSUB
CHUNK = 128


def _sc_deg_body(cpt, n_pad, dst_hbm, zeros_hbm, ones_hbm, out_hbm,
                 idx_v, ones_v, deg_sh):
    cid = lax.axis_index("c")
    sid = lax.axis_index("s")
    wid = cid * NSUB + sid
    rows_t = n_pad // NSUB

    pltpu.sync_copy(ones_hbm, ones_v)
    pltpu.sync_copy(zeros_hbm.at[pl.ds(sid * rows_t, rows_t)],
                    deg_sh.at[pl.ds(sid * rows_t, rows_t)])
    plsc.subcore_barrier()

    def scat(j, _):
        pltpu.sync_copy(dst_hbm.at[wid, j], idx_v)
        pltpu.sync_copy(ones_v, deg_sh.at[idx_v], add=True)
        return 0

    lax.fori_loop(0, cpt, scat, 0)
    plsc.subcore_barrier()

    pltpu.sync_copy(
        deg_sh.at[pl.ds(sid * rows_t, rows_t)],
        out_hbm.at[cid, pl.ds(sid * rows_t, rows_t)],
    )


def _sc_agg_body(cpt, n_pad, y_hbm, src_hbm, dst_hbm, zeros_hbm, out_hbm,
                 sidx_v, didx_v, rows_v, sem, agg_sh):
    cid = lax.axis_index("c")
    sid = lax.axis_index("s")
    wid = cid * NSUB + sid
    rows_t = n_pad // NSUB

    pltpu.sync_copy(zeros_hbm.at[pl.ds(sid * rows_t, rows_t)],
                    agg_sh.at[pl.ds(sid * rows_t, rows_t)])
    plsc.subcore_barrier()

    def loop(j, _):
        pltpu.sync_copy(src_hbm.at[wid, j], sidx_v)
        pltpu.async_copy(y_hbm.at[sidx_v], rows_v, sem).wait()
        pltpu.sync_copy(dst_hbm.at[wid, j], didx_v)
        pltpu.sync_copy(rows_v, agg_sh.at[didx_v], add=True)
        return 0

    lax.fori_loop(0, cpt, loop, 0)
    plsc.subcore_barrier()

    pltpu.sync_copy(
        agg_sh.at[pl.ds(sid * rows_t, rows_t)],
        out_hbm.at[cid, pl.ds(sid * rows_t, rows_t)],
    )


def _tc_pre_body(x_ref, degp_ref, y16_ref, dinv_ref):
    deg = degp_ref[0, :, 0:1] + degp_ref[1, :, 0:1]
    dinv = lax.rsqrt(deg + 1.0)
    dinv16 = jnp.broadcast_to(dinv, x_ref.shape)
    dinv_ref[...] = dinv16
    y16_ref[...] = x_ref[...] * dinv16


def _tc_main_body(n_nodes, hid,
                  y16_ref, aggp_ref, dinv_ref, h0_ref, gW_ref, gb_ref,
                  Wih_ref, Whh_ref, bih_ref, bhh_ref, fW_ref, fb_ref,
                  out_ref, hlast_ref, gi_scr, ys_scr):
    s = dinv_ref[...] * (aggp_ref[0] + aggp_ref[1] + y16_ref[...])
    dn = (((1,), (1,)), ((), ()))
    Wc = lax.dot_general(gW_ref[...], Wih_ref[...], dn,
                         preferred_element_type=jnp.float32)
    bc = lax.dot_general(gb_ref[...], Wih_ref[...], dn,
                         preferred_element_type=jnp.float32) + bih_ref[...]
    gi_scr[...] = jnp.dot(s, Wc, preferred_element_type=jnp.float32) + bc

    Whh = Whh_ref[...].astype(jnp.bfloat16)
    bhh = bhh_ref[...]
    blk = 8
    n_blk = n_nodes // blk

    def step_one(h, gi):
        gh = lax.dot_general(h.astype(jnp.bfloat16), Whh, dn,
                             preferred_element_type=jnp.float32) + bhh
        r = jax.nn.sigmoid(gi[:, 0:hid] + gh[:, 0:hid])
        z = jax.nn.sigmoid(gi[:, hid:2 * hid] + gh[:, hid:2 * hid])
        n = jnp.tanh(gi[:, 2 * hid:] + r * gh[:, 2 * hid:])
        return (1.0 - z) * n + z * h

    def block(b, h):
        base = pl.multiple_of(b * blk, blk)
        gi_blk = gi_scr[pl.ds(base, blk), :]
        rows = []
        for k in range(blk):
            h = step_one(h, gi_blk[k:k + 1, :])
            rows.append(h)
        ys_scr[pl.ds(base, blk), :] = jnp.concatenate(rows, axis=0)
        return h

    h_last = lax.fori_loop(0, n_blk, block, h0_ref[...])
    hlast_ref[...] = h_last
    out_ref[...] = lax.dot_general(
        ys_scr[...], fW_ref[...], dn,
        preferred_element_type=jnp.float32) + fb_ref[...]


def kernel(x, edge_index, hidden_state, gcn_W, gcn_b, W_ih, W_hh, b_ih, b_hh,
           fc_W, fc_b):
    n, in_dim = x.shape
    hid = W_hh.shape[1]
    out_dim = fc_W.shape[0]
    e = edge_index.shape[1]
    f32 = jnp.float32

    n_pad = -(-(n + 1) // (NSUB * CHUNK)) * (NSUB * CHUNK)
    cpt = -(-(-(-e // (NW * CHUNK))) // 8) * 8
    e_pad = NW * cpt * CHUNK
    src = edge_index[0].astype(jnp.int32)
    dst = edge_index[1].astype(jnp.int32)
    pad = jnp.full((e_pad - e,), n, jnp.int32)
    src_r = jnp.concatenate([src, pad]).reshape(NW, cpt, CHUNK)
    dst_r = jnp.concatenate([dst, pad]).reshape(NW, cpt, CHUNK)
    x_pad = jnp.pad(x, ((0, n_pad - n), (0, 0)))
    zeros_h = jnp.zeros((n_pad, 16), f32)
    ones_h = jnp.ones((CHUNK, 16), f32)

    mesh = plsc.VectorSubcoreMesh(core_axis_name="c", subcore_axis_name="s")
    sc_params = pltpu.CompilerParams(use_tc_tiling_on_sc=False)

    sc_deg = pl.kernel(
        functools.partial(_sc_deg_body, cpt, n_pad),
        out_type=jax.ShapeDtypeStruct((NCORE, n_pad, 16), f32),
        mesh=mesh,
        compiler_params=sc_params,
        scratch_types=[
            pltpu.VMEM((CHUNK,), jnp.int32),
            pltpu.VMEM((CHUNK, 16), f32),
            pltpu.VMEM_SHARED((n_pad, 16), f32),
        ],
    )
    degp = sc_deg(dst_r, zeros_h, ones_h)

    y16, dinv16 = pl.pallas_call(
        _tc_pre_body,
        out_shape=[jax.ShapeDtypeStruct((n_pad, in_dim), f32),
                   jax.ShapeDtypeStruct((n_pad, in_dim), f32)],
    )(x_pad, degp)

    sc_agg = pl.kernel(
        functools.partial(_sc_agg_body, cpt, n_pad),
        out_type=jax.ShapeDtypeStruct((NCORE, n_pad, 16), f32),
        mesh=mesh,
        compiler_params=sc_params,
        scratch_types=[
            pltpu.VMEM((CHUNK,), jnp.int32),
            pltpu.VMEM((CHUNK,), jnp.int32),
            pltpu.VMEM((CHUNK, 16), f32),
            pltpu.SemaphoreType.DMA,
            pltpu.VMEM_SHARED((n_pad, 16), f32),
        ],
    )
    aggp = sc_agg(y16, src_r, dst_r, zeros_h)

    out4p, h_last = pl.pallas_call(
        functools.partial(_tc_main_body, n, hid),
        out_shape=[jax.ShapeDtypeStruct((n_pad, out_dim), f32),
                   jax.ShapeDtypeStruct((1, hid), f32)],
        scratch_shapes=[pltpu.VMEM((n_pad, 3 * hid), f32),
                        pltpu.VMEM((n_pad, hid), f32)],
    )(y16, aggp, dinv16, hidden_state.reshape(1, hid), gcn_W,
      gcn_b.reshape(1, hid), W_ih, W_hh, b_ih.reshape(1, 3 * hid),
      b_hh.reshape(1, 3 * hid), fc_W, fc_b.reshape(1, out_dim))

    new_x = jnp.concatenate([x[:, :3], out4p[:n], x[:, 7:8]], axis=1)
    return new_x, h_last.reshape(1, 1, hid)


# speedup vs baseline: 1.0101x; 1.0101x over previous
import functools
import jax
import jax.numpy as jnp
from jax import lax
from jax.experimental import pallas as pl
from jax.experimental.pallas import tpu as pltpu
from jax.experimental.pallas import tpu_sc as plsc

NCORE = 2
NSUB = 16
NW = NCORE * NSUB
CHUNK = 128


def _sc_deg_body(cpt, n_pad, dst_hbm, zeros_hbm, ones_hbm, out_hbm,
                 idx_v, ones_v, deg_sh):
    cid = lax.axis_index("c")
    sid = lax.axis_index("s")
    wid = cid * NSUB + sid
    rows_t = n_pad // NSUB

    pltpu.sync_copy(ones_hbm, ones_v)
    pltpu.sync_copy(zeros_hbm.at[pl.ds(sid * rows_t, rows_t)],
                    deg_sh.at[pl.ds(sid * rows_t, rows_t)])
    plsc.subcore_barrier()

    def scat(j, _):
        pltpu.sync_copy(dst_hbm.at[wid, j], idx_v)
        pltpu.sync_copy(ones_v, deg_sh.at[idx_v], add=True)
        return 0

    lax.fori_loop(0, cpt, scat, 0)
    plsc.subcore_barrier()

    pltpu.sync_copy(
        deg_sh.at[pl.ds(sid * rows_t, rows_t)],
        out_hbm.at[cid, pl.ds(sid * rows_t, rows_t)],
    )


def _sc_agg_body(cpt, n_pad, y_hbm, src_hbm, dst_hbm, zeros_hbm, out_hbm,
                 sidx_v, didx_v, rows_v, sem, agg_sh):
    cid = lax.axis_index("c")
    sid = lax.axis_index("s")
    wid = cid * NSUB + sid
    rows_t = n_pad // NSUB

    pltpu.sync_copy(zeros_hbm.at[pl.ds(sid * rows_t, rows_t)],
                    agg_sh.at[pl.ds(sid * rows_t, rows_t)])
    plsc.subcore_barrier()

    def loop(j, _):
        pltpu.sync_copy(src_hbm.at[wid, j], sidx_v)
        pltpu.async_copy(y_hbm.at[sidx_v], rows_v, sem).wait()
        pltpu.sync_copy(dst_hbm.at[wid, j], didx_v)
        pltpu.sync_copy(rows_v, agg_sh.at[didx_v], add=True)
        return 0

    lax.fori_loop(0, cpt, loop, 0)
    plsc.subcore_barrier()

    pltpu.sync_copy(
        agg_sh.at[pl.ds(sid * rows_t, rows_t)],
        out_hbm.at[cid, pl.ds(sid * rows_t, rows_t)],
    )


def _tc_pre_body(x_ref, degp_ref, y16_ref, dinv_ref):
    deg = degp_ref[0, :, 0:1] + degp_ref[1, :, 0:1]
    dinv = lax.rsqrt(deg + 1.0)
    dinv16 = jnp.broadcast_to(dinv, x_ref.shape)
    dinv_ref[...] = dinv16
    y16_ref[...] = x_ref[...] * dinv16


def _tc_main_body(n_nodes, hid,
                  y16_ref, aggp_ref, dinv_ref, h0_ref, gW_ref, gb_ref,
                  Wih_ref, Whh_ref, bih_ref, bhh_ref, fW_ref, fb_ref,
                  out_ref, hlast_ref, gi_scr, ys_scr):
    s = dinv_ref[...] * (aggp_ref[0] + aggp_ref[1] + y16_ref[...])
    dn = (((1,), (1,)), ((), ()))
    Wc = lax.dot_general(gW_ref[...], Wih_ref[...], dn,
                         preferred_element_type=jnp.float32)
    bhh = bhh_ref[...]
    bc = (lax.dot_general(gb_ref[...], Wih_ref[...], dn,
                          preferred_element_type=jnp.float32) + bih_ref[...]
          + jnp.concatenate([bhh[:, 0:2 * hid],
                             jnp.zeros((1, hid), jnp.float32)], axis=1))
    gi_scr[...] = jnp.dot(s, Wc, preferred_element_type=jnp.float32) + bc

    Whh = Whh_ref[...]
    bhh_n = bhh[:, 2 * hid:]
    blk = 8
    n_blk = n_nodes // blk

    def step_one(h, gi):
        gh = lax.dot_general(h, Whh, dn,
                             preferred_element_type=jnp.float32)
        rz = jax.nn.sigmoid(gi[:, 0:2 * hid] + gh[:, 0:2 * hid])
        r = rz[:, 0:hid]
        z = rz[:, hid:2 * hid]
        n = jnp.tanh(gi[:, 2 * hid:] + r * (gh[:, 2 * hid:] + bhh_n))
        return (1.0 - z) * n + z * h

    def block(b, h):
        base = pl.multiple_of(b * blk, blk)
        gi_blk = gi_scr[pl.ds(base, blk), :]
        rows = []
        for k in range(blk):
            h = step_one(h, gi_blk[k:k + 1, :])
            rows.append(h)
        ys_scr[pl.ds(base, blk), :] = jnp.concatenate(rows, axis=0)
        return h

    h_last = lax.fori_loop(0, n_blk, block, h0_ref[...])
    hlast_ref[...] = h_last
    out_ref[...] = lax.dot_general(
        ys_scr[...], fW_ref[...], dn,
        preferred_element_type=jnp.float32) + fb_ref[...]


def kernel(x, edge_index, hidden_state, gcn_W, gcn_b, W_ih, W_hh, b_ih, b_hh,
           fc_W, fc_b):
    n, in_dim = x.shape
    hid = W_hh.shape[1]
    out_dim = fc_W.shape[0]
    e = edge_index.shape[1]
    f32 = jnp.float32

    n_pad = -(-(n + 1) // (NSUB * CHUNK)) * (NSUB * CHUNK)
    cpt = -(-(-(-e // (NW * CHUNK))) // 8) * 8
    e_pad = NW * cpt * CHUNK
    src = edge_index[0].astype(jnp.int32)
    dst = edge_index[1].astype(jnp.int32)
    pad = jnp.full((e_pad - e,), n, jnp.int32)
    src_r = jnp.concatenate([src, pad]).reshape(NW, cpt, CHUNK)
    dst_r = jnp.concatenate([dst, pad]).reshape(NW, cpt, CHUNK)
    x_pad = jnp.pad(x, ((0, n_pad - n), (0, 0)))
    zeros_h = jnp.zeros((n_pad, 16), f32)
    ones_h = jnp.ones((CHUNK, 16), f32)

    mesh = plsc.VectorSubcoreMesh(core_axis_name="c", subcore_axis_name="s")
    sc_params = pltpu.CompilerParams(use_tc_tiling_on_sc=False)

    sc_deg = pl.kernel(
        functools.partial(_sc_deg_body, cpt, n_pad),
        out_type=jax.ShapeDtypeStruct((NCORE, n_pad, 16), f32),
        mesh=mesh,
        compiler_params=sc_params,
        scratch_types=[
            pltpu.VMEM((CHUNK,), jnp.int32),
            pltpu.VMEM((CHUNK, 16), f32),
            pltpu.VMEM_SHARED((n_pad, 16), f32),
        ],
    )
    degp = sc_deg(dst_r, zeros_h, ones_h)

    y16, dinv16 = pl.pallas_call(
        _tc_pre_body,
        out_shape=[jax.ShapeDtypeStruct((n_pad, in_dim), f32),
                   jax.ShapeDtypeStruct((n_pad, in_dim), f32)],
    )(x_pad, degp)

    sc_agg = pl.kernel(
        functools.partial(_sc_agg_body, cpt, n_pad),
        out_type=jax.ShapeDtypeStruct((NCORE, n_pad, 16), f32),
        mesh=mesh,
        compiler_params=sc_params,
        scratch_types=[
            pltpu.VMEM((CHUNK,), jnp.int32),
            pltpu.VMEM((CHUNK,), jnp.int32),
            pltpu.VMEM((CHUNK, 16), f32),
            pltpu.SemaphoreType.DMA,
            pltpu.VMEM_SHARED((n_pad, 16), f32),
        ],
    )
    aggp = sc_agg(y16, src_r, dst_r, zeros_h)

    out4p, h_last = pl.pallas_call(
        functools.partial(_tc_main_body, n, hid),
        out_shape=[jax.ShapeDtypeStruct((n_pad, out_dim), f32),
                   jax.ShapeDtypeStruct((1, hid), f32)],
        scratch_shapes=[pltpu.VMEM((n_pad, 3 * hid), f32),
                        pltpu.VMEM((n_pad, hid), f32)],
    )(y16, aggp, dinv16, hidden_state.reshape(1, hid), gcn_W,
      gcn_b.reshape(1, hid), W_ih, W_hh, b_ih.reshape(1, 3 * hid),
      b_hh.reshape(1, 3 * hid), fc_W, fc_b.reshape(1, out_dim))

    new_x = jnp.concatenate([x[:, :3], out4p[:n], x[:, 7:8]], axis=1)
    return new_x, h_last.reshape(1, 1, hid)
